# baseline (speedup 1.0000x reference)
import jax
import jax.numpy as jnp
from jax.experimental import pallas as pl
from jax.experimental.pallas import tpu as pltpu
from jax.experimental.pallas import tpu_sc as plsc

_HIDDEN = 64
_SCALE = 8.0
_W = 128


def _emb_body(table_hbm, i_hbm, o_hbm):
    def step(i_vmem, o_vmem):
        pltpu.sync_copy(table_hbm.at[i_vmem.at[0]], o_vmem)

        @pl.loop(0, _W)
        def _(r):
            for c in range(_HIDDEN // 16):
                sl = pl.ds(c * 16, 16)
                o_vmem[r, sl] = o_vmem[r, sl] * _SCALE

    n = i_hbm.shape[1]
    pltpu.emit_pipeline(
        step,
        grid=(n // _W,),
        in_specs=[pl.BlockSpec((1, _W), lambda i: (0, i))],
        out_specs=[pl.BlockSpec((_W, _HIDDEN), lambda i: (i, 0))],
        core_axis_name=("c", "s"),
        dimension_semantics=(pltpu.PARALLEL,),
    )(i_hbm, o_hbm)


def _make_kernel(n):
    mesh = plsc.VectorSubcoreMesh(core_axis_name="c", subcore_axis_name="s")
    return pl.kernel(
        _emb_body,
        out_type=jax.ShapeDtypeStruct((n, _HIDDEN), jnp.float32),
        mesh=mesh,
        compiler_params=pltpu.CompilerParams(use_tc_tiling_on_sc=False),
    )


@jax.jit
def _emb(x_flat, table):
    n = x_flat.shape[1]
    return _make_kernel(n)(table, x_flat)


def kernel(x, table):
    b, s = x.shape
    x_flat = x.reshape(1, b * s).astype(jnp.int32)
    out = _emb(x_flat, table)
    return out.reshape(b, s, _HIDDEN)


# speedup vs baseline: 1.4941x; 1.4941x over previous
import jax
import jax.numpy as jnp
from jax import lax
from jax.experimental import pallas as pl
from jax.experimental.pallas import tpu as pltpu
from jax.experimental.pallas import tpu_sc as plsc

_HIDDEN = 64
_SCALE = 8.0
_NW = 32
_C = 128
_NBUF = 4


def _emb_body(n_per_w, table_hbm, i_hbm, o_hbm, idx_all, rows_v, out_v,
              sem_g, sem_o):
    nchunk = n_per_w // _C
    wid = lax.axis_index("s") * 2 + lax.axis_index("c")
    base = wid * n_per_w

    pltpu.sync_copy(i_hbm.at[pl.ds(base, n_per_w)], idx_all)

    def gather_copy(g, b):
        return pltpu.make_async_copy(
            table_hbm.at[idx_all.at[pl.ds(g * _C, _C)]],
            rows_v.at[b],
            sem_g.at[b],
        )

    def out_copy(g, b):
        return pltpu.make_async_copy(
            out_v.at[b],
            o_hbm.at[pl.ds(base + g * _C, _C)],
            sem_o.at[b],
        )

    for b in range(_NBUF):
        gather_copy(b, b).start()

    @pl.loop(0, nchunk, step=_NBUF)
    def _(g0):
        for b in range(_NBUF):
            g = g0 + b
            gather_copy(g, b).wait()

            @pl.when(g >= _NBUF)
            def _():
                out_copy(g - _NBUF, b).wait()

            @plsc.parallel_loop(0, _C, unroll=8)
            def _(r):
                for c in range(_HIDDEN // 16):
                    sl = pl.ds(c * 16, 16)
                    out_v[b, r, sl] = rows_v[b, r, sl] * _SCALE

            @pl.when(g + _NBUF < nchunk)
            def _():
                gather_copy(g + _NBUF, b).start()

            out_copy(g, b).start()

    for b in range(_NBUF):
        out_copy(nchunk - _NBUF + b, b).wait()


def _make_kernel(n):
    n_per_w = n // _NW
    mesh = plsc.VectorSubcoreMesh(core_axis_name="c", subcore_axis_name="s")

    def body(table_hbm, i_hbm, o_hbm, idx_all, rows_v, out_v, sem_g, sem_o):
        _emb_body(n_per_w, table_hbm, i_hbm, o_hbm, idx_all, rows_v, out_v,
                  sem_g, sem_o)

    return pl.kernel(
        body,
        out_type=jax.ShapeDtypeStruct((n, _HIDDEN), jnp.float32),
        mesh=mesh,
        scratch_types=[
            pltpu.VMEM((n_per_w,), jnp.int32),
            pltpu.VMEM((_NBUF, _C, _HIDDEN), jnp.float32),
            pltpu.VMEM((_NBUF, _C, _HIDDEN), jnp.float32),
            pltpu.SemaphoreType.DMA(_NBUF),
            pltpu.SemaphoreType.DMA(_NBUF),
        ],
        compiler_params=pltpu.CompilerParams(use_tc_tiling_on_sc=False),
    )


@jax.jit
def _emb(x_flat, table):
    n = x_flat.shape[0]
    return _make_kernel(n)(table, x_flat)


def kernel(x, table):
    b, s = x.shape
    x_flat = x.reshape(b * s).astype(jnp.int32)
    out = _emb(x_flat, table)
    return out.reshape(b, s, _HIDDEN)
